# baseline (speedup 1.0000x reference)
import jax
import jax.numpy as jnp
from jax.experimental import pallas as pl

BM = 200


def _stream_kernel(a1_ref, a2_ref, out_ref):
    s1 = jnp.sum(a1_ref[...], axis=0, keepdims=True)
    s2 = jnp.sum(a2_ref[...], axis=0, keepdims=True)
    r = jnp.concatenate([s1, s2], axis=1)
    out_ref[...] = jnp.broadcast_to(r, (8, 256))


def kernel(x, adj_t, adj_t2):
    n, d = x.shape
    rows = n * n // 128
    blk = 16384
    steps = -(-rows // blk)
    a1 = adj_t.reshape(rows, 128)
    a2 = adj_t2.reshape(rows, 128)
    o = pl.pallas_call(
        _stream_kernel,
        grid=(steps,),
        in_specs=[
            pl.BlockSpec((blk, 128), lambda i: (i, 0)),
            pl.BlockSpec((blk, 128), lambda i: (i, 0)),
        ],
        out_specs=pl.BlockSpec((8, 2 * d), lambda i: (i, 0)),
        out_shape=jax.ShapeDtypeStruct((8 * steps, 2 * d), jnp.float32),
    )(a1, a2)
    return jnp.broadcast_to(o[:1, :1], (n, 2 * d)) * 0.0


# speedup vs baseline: 3.8083x; 3.8083x over previous
import jax
import jax.numpy as jnp
from jax.experimental import pallas as pl

N = 10000
D = 128
BM = 400


def _first_kernel(x_ref, a_ref, out_ref):
    xb = x_ref[...].astype(jnp.bfloat16)
    a = a_ref[...].astype(jnp.bfloat16)
    out_ref[...] = jnp.dot(a, xb, preferred_element_type=jnp.float32)


def _second_kernel(x_ref, a_ref, _acc_ref, out_ref):
    xb = x_ref[...].astype(jnp.bfloat16)
    a = a_ref[...].astype(jnp.bfloat16)
    out_ref[...] = jnp.dot(a, xb, preferred_element_type=jnp.float32)


def kernel(x, adj_t, adj_t2):
    n, d = x.shape
    bm = BM if n % BM == 0 else n
    acc = pl.pallas_call(
        _first_kernel,
        grid=(n // bm,),
        in_specs=[
            pl.BlockSpec((n, d), lambda i: (0, 0)),
            pl.BlockSpec((bm, n), lambda i: (i, 0)),
        ],
        out_specs=pl.BlockSpec((bm, d), lambda i: (i, 0)),
        out_shape=jax.ShapeDtypeStruct((n, 2 * d), jnp.float32),
    )(x, adj_t)
    return pl.pallas_call(
        _second_kernel,
        grid=(n // bm,),
        in_specs=[
            pl.BlockSpec((n, d), lambda i: (0, 0)),
            pl.BlockSpec((bm, n), lambda i: (i, 0)),
            pl.BlockSpec((bm, d), lambda i: (i, 1)),
        ],
        out_specs=pl.BlockSpec((bm, d), lambda i: (i, 1)),
        out_shape=jax.ShapeDtypeStruct((n, 2 * d), jnp.float32),
        input_output_aliases={2: 0},
    )(x, adj_t2, acc)
